# Initial kernel scaffold; baseline (speedup 1.0000x reference)
import jax
import jax.numpy as jnp
from jax.experimental import pallas as pl


def kernel(xyz1, xyz2, points1, points2, w0, b0, g0, be0, w1, b1, g1, be1):
    raise NotImplementedError("write your pallas kernel here")


# speedup vs baseline: 38.9784x; 38.9784x over previous
import jax
import jax.numpy as jnp
from jax.experimental import pallas as pl

B, N, S, D1, D2 = 4, 8192, 2048, 64, 128
C0, C1 = 128, 128
NB1 = 512
NB2 = 1024
EPS = 1e-5


def _k1_body(x1_ref, x2_ref, p1_ref, p2_ref, w0a_ref, w0b_ref, b0_ref,
             h0_ref, s_ref, ss_ref):
    b = pl.program_id(0)
    j = pl.program_id(1)

    x1 = x1_ref[0]
    x2 = x2_ref[0]
    p1 = p1_ref[0]
    p2 = p2_ref[0]

    sq1 = jnp.sum(x1 * x1, axis=0)
    sq2 = jnp.sum(x2 * x2, axis=0)
    d = -2.0 * jax.lax.dot_general(
        x1, x2, (((0,), (0,)), ((), ())),
        preferred_element_type=jnp.float32)
    d = d + sq1[:, None] + sq2[None, :]

    iota = jax.lax.broadcasted_iota(jnp.int32, d.shape, 1)
    dm = d
    wnum = jnp.zeros(d.shape, jnp.float32)
    recsum = jnp.zeros((d.shape[0], 1), jnp.float32)
    for _ in range(3):
        m = jnp.min(dm, axis=1, keepdims=True)
        hit = dm == m
        i = jnp.min(jnp.where(hit, iota, S), axis=1, keepdims=True)
        sel = iota == i
        rec = 1.0 / (m + 1e-8)
        wnum = jnp.where(sel, rec, wnum)
        recsum = recsum + rec
        dm = jnp.where(sel, jnp.float32(jnp.inf), dm)

    itp = jax.lax.dot_general(
        wnum, p2, (((1,), (1,)), ((), ())),
        preferred_element_type=jnp.float32)
    itp = itp * (1.0 / recsum)

    h0 = jax.lax.dot_general(
        w0a_ref[...], p1, (((1,), (0,)), ((), ())),
        preferred_element_type=jnp.float32)
    h0 = h0 + jax.lax.dot_general(
        w0b_ref[...], itp, (((1,), (1,)), ((), ())),
        preferred_element_type=jnp.float32)
    h0 = h0 + b0_ref[...]
    h0_ref[0] = h0

    @pl.when((b == 0) & (j == 0))
    def _init():
        s_ref[...] = jnp.zeros_like(s_ref)
        ss_ref[...] = jnp.zeros_like(ss_ref)

    s_ref[...] += jnp.sum(h0, axis=1, keepdims=True)
    ss_ref[...] += jnp.sum(h0 * h0, axis=1, keepdims=True)


def _k2_body(h0_ref, a_ref, c_ref, w1_ref, b1_ref, h1_ref, s_ref, ss_ref):
    b = pl.program_id(0)
    j = pl.program_id(1)
    z = jnp.maximum(h0_ref[0] * a_ref[...] + c_ref[...], 0.0)
    h1 = jax.lax.dot_general(
        w1_ref[...], z, (((1,), (0,)), ((), ())),
        preferred_element_type=jnp.float32) + b1_ref[...]
    h1_ref[0] = h1

    @pl.when((b == 0) & (j == 0))
    def _init():
        s_ref[...] = jnp.zeros_like(s_ref)
        ss_ref[...] = jnp.zeros_like(ss_ref)

    s_ref[...] += jnp.sum(h1, axis=1, keepdims=True)
    ss_ref[...] += jnp.sum(h1 * h1, axis=1, keepdims=True)


def _k3_body(h1_ref, a_ref, c_ref, out_ref):
    out_ref[0] = jnp.maximum(h1_ref[0] * a_ref[...] + c_ref[...], 0.0)


def kernel(xyz1, xyz2, points1, points2, w0, b0, g0, be0, w1, b1, g1, be1):
    f32 = jnp.float32
    w0a = w0[:, :D1]
    w0b = w0[:, D1:]
    col = lambda v: v.reshape(-1, 1).astype(f32)

    h0, s0, ss0 = pl.pallas_call(
        _k1_body,
        grid=(B, N // NB1),
        in_specs=[
            pl.BlockSpec((1, 3, NB1), lambda b, j: (b, 0, j)),
            pl.BlockSpec((1, 3, S), lambda b, j: (b, 0, 0)),
            pl.BlockSpec((1, D1, NB1), lambda b, j: (b, 0, j)),
            pl.BlockSpec((1, D2, S), lambda b, j: (b, 0, 0)),
            pl.BlockSpec((C0, D1), lambda b, j: (0, 0)),
            pl.BlockSpec((C0, D2), lambda b, j: (0, 0)),
            pl.BlockSpec((C0, 1), lambda b, j: (0, 0)),
        ],
        out_specs=[
            pl.BlockSpec((1, C0, NB1), lambda b, j: (b, 0, j)),
            pl.BlockSpec((C0, 1), lambda b, j: (0, 0)),
            pl.BlockSpec((C0, 1), lambda b, j: (0, 0)),
        ],
        out_shape=[
            jax.ShapeDtypeStruct((B, C0, N), f32),
            jax.ShapeDtypeStruct((C0, 1), f32),
            jax.ShapeDtypeStruct((C0, 1), f32),
        ],
    )(xyz1, xyz2, points1, points2, w0a, w0b, col(b0))

    n = float(B * N)
    mean0 = s0 / n
    var0 = ss0 / n - mean0 * mean0
    a0 = col(g0) * jax.lax.rsqrt(var0 + EPS)
    c0 = col(be0) - mean0 * a0

    h1, s1, ss1 = pl.pallas_call(
        _k2_body,
        grid=(B, N // NB2),
        in_specs=[
            pl.BlockSpec((1, C0, NB2), lambda b, j: (b, 0, j)),
            pl.BlockSpec((C0, 1), lambda b, j: (0, 0)),
            pl.BlockSpec((C0, 1), lambda b, j: (0, 0)),
            pl.BlockSpec((C1, C0), lambda b, j: (0, 0)),
            pl.BlockSpec((C1, 1), lambda b, j: (0, 0)),
        ],
        out_specs=[
            pl.BlockSpec((1, C1, NB2), lambda b, j: (b, 0, j)),
            pl.BlockSpec((C1, 1), lambda b, j: (0, 0)),
            pl.BlockSpec((C1, 1), lambda b, j: (0, 0)),
        ],
        out_shape=[
            jax.ShapeDtypeStruct((B, C1, N), f32),
            jax.ShapeDtypeStruct((C1, 1), f32),
            jax.ShapeDtypeStruct((C1, 1), f32),
        ],
    )(h0, a0, c0, w1, col(b1))

    mean1 = s1 / n
    var1 = ss1 / n - mean1 * mean1
    a1 = col(g1) * jax.lax.rsqrt(var1 + EPS)
    c1 = col(be1) - mean1 * a1

    out = pl.pallas_call(
        _k3_body,
        grid=(B, N // NB2),
        in_specs=[
            pl.BlockSpec((1, C1, NB2), lambda b, j: (b, 0, j)),
            pl.BlockSpec((C1, 1), lambda b, j: (0, 0)),
            pl.BlockSpec((C1, 1), lambda b, j: (0, 0)),
        ],
        out_specs=pl.BlockSpec((1, C1, NB2), lambda b, j: (b, 0, j)),
        out_shape=jax.ShapeDtypeStruct((B, C1, N), f32),
    )(h1, a1, c1)

    return out
